# baseline (speedup 1.0000x reference)
import functools

import jax
import jax.numpy as jnp
from jax import lax
from jax.experimental import pallas as pl
from jax.experimental.pallas import tpu as pltpu
from jax.experimental.pallas import tpu_sc as plsc

_B, _L, _EMB = 4096, 128, 64
_NSEG = 3
_NC, _NS = 2, 16
_NW = _NC * _NS
_SEQ_W = _B // _NW
_NBUF = 4

_mesh = plsc.VectorSubcoreMesh(
    core_axis_name="c", subcore_axis_name="s",
    num_cores=_NC, num_subcores=_NS)


@functools.partial(
    pl.kernel,
    out_type=jax.ShapeDtypeStruct((_B * _L, _EMB), jnp.float32),
    mesh=_mesh,
    scratch_types=[
        pltpu.VMEM((_SEQ_W, _L), jnp.int32),
        pltpu.VMEM((_SEQ_W, _L), jnp.int32),
        pltpu.VMEM((_L, _EMB), jnp.float32),
        pltpu.VMEM((_NBUF * _L, _EMB), jnp.float32),
        pltpu.SemaphoreType.DMA,
    ] + [pltpu.SemaphoreType.DMA] * (3 * _NBUF),
    compiler_params=pltpu.CompilerParams(use_tc_tiling_on_sc=False),
)
def _emb_kernel(ids_hbm, sid_hbm, word_hbm, pos_hbm, seg_hbm, out_hbm,
                idx_v, sid_v, pos_v, rows_v, psem, *sems):
    gsem = sems[:_NBUF]
    hsem = sems[_NBUF:2 * _NBUF]
    ssem = sems[2 * _NBUF:]
    w = lax.axis_index("s") * _NC + lax.axis_index("c")
    base_seq = w * _SEQ_W

    pltpu.sync_copy(ids_hbm.at[pl.ds(base_seq, _SEQ_W)], idx_v)
    pltpu.sync_copy(sid_hbm.at[pl.ds(base_seq, _SEQ_W)], sid_v)
    pltpu.sync_copy(pos_hbm, pos_v)

    def rows_buf(b):
        return rows_v.at[pl.ds(b * _L, _L)]

    def word_gather(g, b):
        return pltpu.make_async_copy(
            word_hbm.at[idx_v.at[g]], rows_buf(b), gsem[b])

    def seg_gather(g, b):
        return pltpu.make_async_copy(
            seg_hbm.at[sid_v.at[g]], rows_buf(b), hsem[b])

    def scatter(g, b):
        return pltpu.make_async_copy(
            rows_buf(b), out_hbm.at[pl.ds((base_seq + g) * _L, _L)], ssem[b])

    def fill(g, b):
        pltpu.async_copy(pos_hbm, rows_buf(b), psem).wait()
        word_gather(g, b).start(add=True)
        seg_gather(g, b).start(add=True)

    for b in range(_NBUF - 1):
        fill(b, b)

    def step(s, carry):
        for b in range(_NBUF):
            g = s * _NBUF + b
            word_gather(g, b).wait()
            seg_gather(g, b).wait()
            scatter(g, b).start()

            bp = (b + _NBUF - 1) % _NBUF

            @pl.when(g >= 1)
            def _wait_prev():
                scatter(g - 1, bp).wait()

            @pl.when(g + _NBUF - 1 < _SEQ_W)
            def _prefetch():
                fill(g + _NBUF - 1, bp)
        return carry
    lax.fori_loop(0, _SEQ_W // _NBUF, step, 0)

    scatter(_SEQ_W - 1, _NBUF - 1).wait()


def kernel(input_ids, segment_ids, word_table, pos_table, seg_table):
    ids = jnp.asarray(input_ids, jnp.int32)
    sids = jnp.asarray(segment_ids, jnp.int32)
    out = _emb_kernel(ids, sids, word_table, pos_table, seg_table)
    return out.reshape(_B, _L, _EMB)


# speedup vs baseline: 10.3251x; 10.3251x over previous
import functools

import jax
import jax.numpy as jnp
from jax import lax
from jax.experimental import pallas as pl
from jax.experimental.pallas import tpu as pltpu
from jax.experimental.pallas import tpu_sc as plsc

_B, _L, _EMB = 4096, 128, 64
_NSEG = 3
_NC, _NS = 2, 16
_NW = _NC * _NS
_SEQ_W = _B // _NW
_EV = _EMB // 16
_NBUF = 4

_mesh = plsc.VectorSubcoreMesh(
    core_axis_name="c", subcore_axis_name="s",
    num_cores=_NC, num_subcores=_NS)


@functools.partial(
    pl.kernel,
    out_type=jax.ShapeDtypeStruct((_B, _L, _EMB), jnp.float32),
    mesh=_mesh,
    scratch_types=[
        pltpu.VMEM((_SEQ_W, _L), jnp.int32),
        pltpu.VMEM((_SEQ_W, _L), jnp.int32),
        pltpu.VMEM((_L, _EMB), jnp.float32),
        pltpu.VMEM((_NSEG, _EMB), jnp.float32),
        pltpu.VMEM((_NSEG * _L, _EMB), jnp.float32),
        pltpu.VMEM((_NBUF * _L, _EMB), jnp.float32),
    ] + [pltpu.SemaphoreType.DMA] * (2 * _NBUF),
    compiler_params=pltpu.CompilerParams(use_tc_tiling_on_sc=False),
)
def _emb_kernel(ids_hbm, sid_hbm, word_hbm, pos_hbm, seg_hbm, out_hbm,
                idx_v, sid_v, pos_v, seg_v, combo_v, rows_v, *sems):
    gsem = sems[:_NBUF]
    ssem = sems[_NBUF:]
    w = lax.axis_index("s") * _NC + lax.axis_index("c")
    base_seq = w * _SEQ_W

    pltpu.sync_copy(ids_hbm.at[pl.ds(base_seq, _SEQ_W)], idx_v)
    pltpu.sync_copy(sid_hbm.at[pl.ds(base_seq, _SEQ_W)], sid_v)
    pltpu.sync_copy(pos_hbm, pos_v)
    pltpu.sync_copy(seg_hbm, seg_v)

    def build(r, carry):
        l = lax.rem(r, _L)
        s = lax.div(r, _L)
        for e in range(_EV):
            combo_v[r, pl.ds(e * 16, 16)] = (
                pos_v[l, pl.ds(e * 16, 16)] + seg_v[s, pl.ds(e * 16, 16)])
        return carry
    lax.fori_loop(0, _NSEG * _L, build, 0)

    def rows_buf(b):
        return rows_v.at[pl.ds(b * _L, _L)]

    def gather(g, b):
        return pltpu.make_async_copy(
            word_hbm.at[idx_v.at[g]], rows_buf(b), gsem[b])

    def scatter(g, b):
        return pltpu.make_async_copy(
            rows_buf(b), out_hbm.at[base_seq + g], ssem[b])

    for b in range(_NBUF - 1):
        gather(b, b).start()

    def step(s, carry):
        for b in range(_NBUF):
            g = s * _NBUF + b
            gather(g, b).wait()

            def tok16(grp, c2):
                base = b * _L
                sv = sid_v[g, pl.ds(grp * 16, 16)]
                for j in range(16):
                    i = grp * 16 + j
                    cb = sv[j] * _L + i
                    for e in range(_EV):
                        rows_v[base + i, pl.ds(e * 16, 16)] = (
                            rows_v[base + i, pl.ds(e * 16, 16)]
                            + combo_v[cb, pl.ds(e * 16, 16)])
                return c2
            lax.fori_loop(0, _L // 16, tok16, 0)

            scatter(g, b).start()

            bp = (b + _NBUF - 1) % _NBUF

            @pl.when(g >= 1)
            def _wait_prev():
                scatter(g - 1, bp).wait()

            @pl.when(g + _NBUF - 1 < _SEQ_W)
            def _prefetch():
                gather(g + _NBUF - 1, bp).start()
        return carry
    lax.fori_loop(0, _SEQ_W // _NBUF, step, 0)

    scatter(_SEQ_W - 1, _NBUF - 1).wait()


def kernel(input_ids, segment_ids, word_table, pos_table, seg_table):
    ids = jnp.asarray(input_ids, jnp.int32)
    sids = jnp.asarray(segment_ids, jnp.int32)
    return _emb_kernel(ids, sids, word_table, pos_table, seg_table)
